# baseline (speedup 1.0000x reference)
import jax
import jax.numpy as jnp
from jax import lax
from jax.experimental import pallas as pl
from jax.experimental.pallas import tpu as pltpu, tpu_sc as plsc

N, C = 16384, 1000
BC = 2048
SCW = 4096
CS = N - SCW
NTC = CS // BC
NC_, NS_, L = 2, 16, 16
NWRK = NC_ * NS_
CW = SCW // NWRK
ROWS = 40
NCH = C // ROWS
NV = CW // L


def _tc_body(tgt_ref, rwd_ref, pt_ref, out_ref):
    g = pl.program_id(0)
    tgt = tgt_ref[...]
    rwd = rwd_ref[...]
    pb = pt_ref[...]
    rows = lax.broadcasted_iota(jnp.int32, (C, BC), 0)
    picked = jnp.where(rows == tgt[None, :], pb, 0.0)
    partial = jnp.sum(jnp.sum(picked, axis=0) * rwd)

    @pl.when(g == 0)
    def _():
        out_ref[0, 0] = 0.0

    out_ref[0, 0] += partial * (-1.0 / N)


def _sc_body(pt_hbm, tgt_hbm, rwd_hbm, out_hbm, tgt_v, rwd_v, acc_v, buf,
             out16, sems):
    c = lax.axis_index("c")
    s = lax.axis_index("s")
    wid = s * NC_ + c
    c0 = pl.multiple_of(CS + wid * CW, 128)
    pltpu.sync_copy(tgt_hbm.at[pl.ds(c0, CW)], tgt_v)
    pltpu.sync_copy(rwd_hbm.at[pl.ds(c0, CW)], rwd_v)
    for v in range(NV):
        acc_v[pl.ds(v * L, L)] = jnp.zeros((L,), jnp.float32)

    def start(ch, p):
        pltpu.make_async_copy(
            pt_hbm.at[pl.ds(pl.multiple_of(ch * ROWS, 8), ROWS),
                      pl.ds(c0, CW)],
            buf.at[p],
            sems.at[p],
        ).start()

    def wait(p):
        pltpu.make_async_copy(
            pt_hbm.at[pl.ds(0, ROWS), pl.ds(0, CW)], buf.at[0], sems.at[p]
        ).wait()

    start(0, 0)

    def step(ch, _):
        p = ch & 1

        @pl.when(ch + 1 < NCH)
        def _():
            start(ch + 1, 1 - p)

        wait(p)
        for v in range(NV):
            pv = acc_v[pl.ds(v * L, L)]
            tvec = tgt_v[pl.ds(v * L, L)]
            for rr in range(ROWS):
                j = ch * ROWS + rr
                pb = buf[p, rr, pl.ds(v * L, L)]
                pv = jnp.where(tvec == j, pb, pv)
            acc_v[pl.ds(v * L, L)] = pv
        return 0

    lax.fori_loop(0, NCH, step, 0, unroll=False)

    acc = jnp.zeros((L,), jnp.float32)
    for v in range(NV):
        acc = acc + acc_v[pl.ds(v * L, L)] * rwd_v[pl.ds(v * L, L)]
    out16[...] = acc * (-1.0 / N)
    pltpu.sync_copy(out16, out_hbm.at[wid])


@jax.jit
def _ganloss(pt, target, reward):
    mesh = plsc.VectorSubcoreMesh(core_axis_name="c", subcore_axis_name="s")
    sc_run = pl.kernel(
        _sc_body,
        out_type=jax.ShapeDtypeStruct((NWRK, L), jnp.float32),
        mesh=mesh,
        scratch_types=[
            pltpu.VMEM((CW,), jnp.int32),
            pltpu.VMEM((CW,), jnp.float32),
            pltpu.VMEM((CW,), jnp.float32),
            pltpu.VMEM((2, ROWS, CW), jnp.float32),
            pltpu.VMEM((L,), jnp.float32),
            pltpu.SemaphoreType.DMA((2,)),
        ],
    )
    sc_out = sc_run(pt, target, reward)

    tc_out = pl.pallas_call(
        _tc_body,
        grid=(NTC,),
        in_specs=[
            pl.BlockSpec((BC,), lambda g: (g,)),
            pl.BlockSpec((BC,), lambda g: (g,)),
            pl.BlockSpec((C, BC), lambda g: (0, g)),
        ],
        out_specs=pl.BlockSpec(
            (1, 1), lambda g: (0, 0), memory_space=pltpu.SMEM
        ),
        out_shape=jax.ShapeDtypeStruct((1, 1), jnp.float32),
    )(target, reward, pt)
    return tc_out[0, 0] + jnp.sum(sc_out)


def kernel(prob, target, reward):
    return _ganloss(prob.T, target.astype(jnp.int32), reward)


# speedup vs baseline: 1.8988x; 1.8988x over previous
import jax
import jax.numpy as jnp
from jax import lax
from jax.experimental import pallas as pl
from jax.experimental.pallas import tpu as pltpu

N, C = 16384, 1000
BR = 200
GRID = C // BR


def _body(tgt_ref, rwd_ref, pt_ref, out_ref, acc_ref):
    g = pl.program_id(0)
    tgt = tgt_ref[...]
    pb = pt_ref[...]
    rows = g * BR + lax.broadcasted_iota(jnp.int32, (BR, N), 0)
    picked = jnp.sum(jnp.where(rows == tgt[None, :], pb, 0.0), axis=0,
                     keepdims=True)

    @pl.when(g == 0)
    def _():
        acc_ref[...] = jnp.zeros((1, N), jnp.float32)

    acc_ref[...] += picked

    @pl.when(g == GRID - 1)
    def _():
        rwd = rwd_ref[...]
        out_ref[0, 0] = jnp.sum(acc_ref[0, :] * rwd) * (-1.0 / N)


@jax.jit
def _ganloss(pt, target, reward):
    out = pl.pallas_call(
        _body,
        grid=(GRID,),
        in_specs=[
            pl.BlockSpec((N,), lambda g: (0,)),
            pl.BlockSpec((N,), lambda g: (0,)),
            pl.BlockSpec((BR, N), lambda g: (g, 0)),
        ],
        out_specs=pl.BlockSpec(
            (1, 1), lambda g: (0, 0), memory_space=pltpu.SMEM
        ),
        out_shape=jax.ShapeDtypeStruct((1, 1), jnp.float32),
        scratch_shapes=[pltpu.VMEM((1, N), jnp.float32)],
    )(target, reward, pt)
    return out[0, 0]


def kernel(prob, target, reward):
    return _ganloss(prob.T, target.astype(jnp.int32), reward)
